# baseline (speedup 1.0000x reference)
import numpy as np
import jax
import jax.numpy as jnp
from jax.experimental import pallas as pl

_HIDDEN = 32
_LAYER_SIZES = [128, 128, 64, 64]
_HEADS = [8, 8, 16, 16, 1]
_BATCH = 16
_NUM_NODES = 6250
_DIMS = [1] + _LAYER_SIZES + [_HIDDEN]

_W = 8
_R = _BATCH * _W


def _one_hot_heads(h, hc):
    lane = jax.lax.broadcasted_iota(jnp.int32, (h, hc), 1)
    head = jax.lax.broadcasted_iota(jnp.int32, (h, hc), 0)
    return (lane // (hc // h) == head).astype(jnp.bfloat16)


def _shift_down(a):
    return jnp.concatenate([a[a.shape[0] - 1:], a[: a.shape[0] - 1]], axis=0)


def _shift_up(a):
    return jnp.concatenate([a[1:], a[:1]], axis=0)


def _mdot(a, b):
    return jnp.dot(a.astype(jnp.bfloat16), b.astype(jnp.bfloat16),
                   preferred_element_type=jnp.float32)


def _wfold(msg_l, msg_r, wl, wr, h, c):
    R = msg_l.shape[0]
    if h == 1:
        return wl * msg_l + wr * msg_r
    if c == 128:
        acc = None
        for k in range(h):
            t = (wl[:, k:k + 1] * msg_l[:, k * 128:(k + 1) * 128]
                 + wr[:, k:k + 1] * msg_r[:, k * 128:(k + 1) * 128])
            acc = t if acc is None else acc + t
    else:
        acc128 = None
        for k in range(h // 2):
            pwl = jnp.concatenate(
                [jnp.broadcast_to(wl[:, 2 * k:2 * k + 1], (R, 64)),
                 jnp.broadcast_to(wl[:, 2 * k + 1:2 * k + 2], (R, 64))], axis=1)
            pwr = jnp.concatenate(
                [jnp.broadcast_to(wr[:, 2 * k:2 * k + 1], (R, 64)),
                 jnp.broadcast_to(wr[:, 2 * k + 1:2 * k + 2], (R, 64))], axis=1)
            t = pwl * msg_l[:, k * 128:(k + 1) * 128] + pwr * msg_r[:, k * 128:(k + 1) * 128]
            acc128 = t if acc128 is None else acc128 + t
        acc = acc128[:, :64] + acc128[:, 64:]
    return acc * (1.0 / h)


def _conv(x, Wm, bm, attf, E, h, c, mask_r):
    f32 = jnp.float32
    if x.shape[1] == 1:
        msg = x * Wm + bm
    else:
        msg = _mdot(x, Wm) + bm
    prod = msg * attf
    if h == 1:
        s = jnp.sum(prod, axis=1, keepdims=True)
    else:
        ph = prod.astype(jnp.bfloat16)
        plo = (prod - ph.astype(f32)).astype(jnp.bfloat16)
        Eb = E
        dims = (((1,), (1,)), ((), ()))
        s = (jax.lax.dot_general(ph, Eb, dims, preferred_element_type=f32)
             + jax.lax.dot_general(plo, Eb, dims, preferred_element_type=f32))
    s = jnp.where(s >= 0, s, 0.2 * s)
    sl = _shift_down(s)
    sr = jnp.where(mask_r, _shift_up(s), -1e30)
    m = jnp.maximum(sl, sr)
    exl = jnp.exp(sl - m)
    exr = jnp.where(mask_r, jnp.exp(sr - m), 0.0)
    den = exl + exr + 1e-16
    return _wfold(_shift_down(msg), _shift_up(msg),
                  exl / den, exr / den, h, c)


def _body(x_ref, Wm0, Ws0, Wm1, Wm2, Ws2, Wm3, Wm4, Ws4, Wc, sm_ref,
          critic_ref, feat_ref):
    f32 = jnp.float32
    row = jax.lax.broadcasted_iota(jnp.int32, (_R, 1), 0)
    mask_r = (row % _W) != (_W - 1)
    E8 = _one_hot_heads(8, 1024)
    E16 = _one_hot_heads(16, 1024)

    S = sm_ref[...]
    bm0, a0 = S[0:1], S[1:2]
    bs0 = S[2:3, :128]
    bm1, a1 = S[3:4], S[4:5]
    bm2, a2 = S[5:6], S[6:7]
    bs2 = S[7:8, :64]
    bm3, a3 = S[8:9], S[9:10]
    bm4, a4 = S[10:11, :32], S[11:12, :32]
    bs4 = S[12:13, :32]
    bc = S[13:14, :1]

    x = x_ref[...]
    layers = (
        (Wm0, bm0, a0, E8, 8, 128, Ws0, bs0),
        (Wm1, bm1, a1, E8, 8, 128, None, None),
        (Wm2, bm2, a2, E16, 16, 64, Ws2, bs2),
        (Wm3, bm3, a3, E16, 16, 64, None, None),
        (Wm4, bm4, a4, None, 1, 32, Ws4, bs4),
    )
    for li, (Wm, bm, attf, E, h, c, Ws, bs) in enumerate(layers):
        out = _conv(x, Wm[...], bm, attf, E, h, c, mask_r)
        if Ws is not None:
            if x.shape[1] == 1:
                out = out + x * Ws[...] + bs
            else:
                out = out + _mdot(x, Ws[...]) + bs
        else:
            out = out + x
        x = jnp.where(out > 0, out, jnp.exp(jnp.minimum(out, 0.0)) - 1.0) if li < 4 else out

    lane = jax.lax.broadcasted_iota(jnp.int32, (_BATCH, _R), 1)
    brow = jax.lax.broadcasted_iota(jnp.int32, (_BATCH, _R), 0)
    G = (lane == brow * _W + (_W - 1)).astype(f32)
    feat = jnp.dot(G, x, precision=jax.lax.Precision.HIGHEST,
                   preferred_element_type=f32)
    feat_ref[...] = feat
    critic_ref[...] = _mdot(feat, Wc[...]) + bc


def kernel(nodes, W_msg0, b_msg0, att0, W_self0, b_self0,
           W_msg1, b_msg1, att1,
           W_msg2, b_msg2, att2, W_self2, b_self2,
           W_msg3, b_msg3, att3,
           W_msg4, b_msg4, att4, W_self4, b_self4,
           W_critic, b_critic):
    f32 = jnp.float32
    x0 = nodes[:, _NUM_NODES - _W:].reshape(_R, 1).astype(f32)

    def _row(v):
        v = v.reshape(-1).astype(f32)
        return jnp.pad(v, (0, 1024 - v.shape[0]))

    smalls = jnp.stack([
        _row(b_msg0), _row(att0), _row(b_self0),
        _row(b_msg1), _row(att1),
        _row(b_msg2), _row(att2), _row(b_self2),
        _row(b_msg3), _row(att3),
        _row(b_msg4), _row(att4), _row(b_self4),
        _row(b_critic), _row(b_critic), _row(b_critic),
    ])
    args = [
        x0,
        W_msg0, W_self0, W_msg1, W_msg2, W_self2, W_msg3,
        W_msg4, W_self4, W_critic, smalls,
    ]
    critic, feat = pl.pallas_call(
        _body,
        out_shape=(
            jax.ShapeDtypeStruct((_BATCH, 1), f32),
            jax.ShapeDtypeStruct((_BATCH, _HIDDEN), f32),
        ),
    )(*args)
    return critic, feat


# speedup vs baseline: 1.1721x; 1.1721x over previous
import numpy as np
import jax
import jax.numpy as jnp
from jax.experimental import pallas as pl

_HIDDEN = 32
_LAYER_SIZES = [128, 128, 64, 64]
_HEADS = [8, 8, 16, 16, 1]
_BATCH = 16
_NUM_NODES = 6250
_DIMS = [1] + _LAYER_SIZES + [_HIDDEN]

_W = 8
_R = _BATCH * _W


def _one_hot_heads(h, hc):
    lane = jax.lax.broadcasted_iota(jnp.int32, (h, hc), 1)
    head = jax.lax.broadcasted_iota(jnp.int32, (h, hc), 0)
    return (lane // (hc // h) == head).astype(jnp.bfloat16)


def _shift_down(a):
    return jnp.concatenate([a[a.shape[0] - 1:], a[: a.shape[0] - 1]], axis=0)


def _shift_up(a):
    return jnp.concatenate([a[1:], a[:1]], axis=0)


def _mdot(a, b):
    return jnp.dot(a.astype(jnp.bfloat16), b.astype(jnp.bfloat16),
                   preferred_element_type=jnp.float32)


def _wfold(msg_l, msg_r, wl, wr, h, c):
    R = msg_l.shape[0]
    if h == 1:
        return wl * msg_l + wr * msg_r
    if c == 128:
        acc = None
        for k in range(h):
            t = (wl[:, k:k + 1] * msg_l[:, k * 128:(k + 1) * 128]
                 + wr[:, k:k + 1] * msg_r[:, k * 128:(k + 1) * 128])
            acc = t if acc is None else acc + t
    else:
        acc128 = None
        for k in range(h // 2):
            pwl = jnp.concatenate(
                [jnp.broadcast_to(wl[:, 2 * k:2 * k + 1], (R, 64)),
                 jnp.broadcast_to(wl[:, 2 * k + 1:2 * k + 2], (R, 64))], axis=1)
            pwr = jnp.concatenate(
                [jnp.broadcast_to(wr[:, 2 * k:2 * k + 1], (R, 64)),
                 jnp.broadcast_to(wr[:, 2 * k + 1:2 * k + 2], (R, 64))], axis=1)
            t = pwl * msg_l[:, k * 128:(k + 1) * 128] + pwr * msg_r[:, k * 128:(k + 1) * 128]
            acc128 = t if acc128 is None else acc128 + t
        acc = acc128[:, :64] + acc128[:, 64:]
    return acc * (1.0 / h)


def _conv(x, Wm, bm, attf, E, h, c, mask_r):
    f32 = jnp.float32
    if x.shape[1] == 1:
        msg = x * Wm + bm
    else:
        msg = _mdot(x, Wm) + bm
    prod = msg * attf
    if h == 1:
        s = jnp.sum(prod, axis=1, keepdims=True)
    else:
        ph = prod.astype(jnp.bfloat16)
        plo = (prod - ph.astype(f32)).astype(jnp.bfloat16)
        Eb = E
        dims = (((1,), (1,)), ((), ()))
        s = (jax.lax.dot_general(ph, Eb, dims, preferred_element_type=f32)
             + jax.lax.dot_general(plo, Eb, dims, preferred_element_type=f32))
    s = jnp.where(s >= 0, s, 0.2 * s)
    sl = _shift_down(s)
    sr = jnp.where(mask_r, _shift_up(s), -1e30)
    m = jnp.maximum(sl, sr)
    exl = jnp.exp(sl - m)
    exr = jnp.where(mask_r, jnp.exp(sr - m), 0.0)
    den = exl + exr + 1e-16
    return _wfold(_shift_down(msg), _shift_up(msg),
                  exl / den, exr / den, h, c)


def _body(x_ref,
          Wm0, bm0, a0, Ws0, bs0,
          Wm1, bm1, a1,
          Wm2, bm2, a2, Ws2, bs2,
          Wm3, bm3, a3,
          Wm4, bm4, a4, Ws4, bs4,
          Wc, bc,
          critic_ref, feat_ref):
    f32 = jnp.float32
    row = jax.lax.broadcasted_iota(jnp.int32, (_R, 1), 0)
    mask_r = (row % _W) != (_W - 1)
    E8 = _one_hot_heads(8, 1024)
    E16 = _one_hot_heads(16, 1024)

    x = x_ref[...]
    layers = (
        (Wm0, bm0, a0, E8, 8, 128, Ws0, bs0),
        (Wm1, bm1, a1, E8, 8, 128, None, None),
        (Wm2, bm2, a2, E16, 16, 64, Ws2, bs2),
        (Wm3, bm3, a3, E16, 16, 64, None, None),
        (Wm4, bm4, a4, None, 1, 32, Ws4, bs4),
    )
    for li, (Wm, bm, attf, E, h, c, Ws, bs) in enumerate(layers):
        out = _conv(x, Wm[...], bm[...], attf[...], E, h, c, mask_r)
        if Ws is not None:
            if x.shape[1] == 1:
                out = out + x * Ws[...] + bs[...]
            else:
                out = out + _mdot(x, Ws[...]) + bs[...]
        else:
            out = out + x
        x = jnp.where(out > 0, out, jnp.exp(jnp.minimum(out, 0.0)) - 1.0) if li < 4 else out

    lane = jax.lax.broadcasted_iota(jnp.int32, (_BATCH, _R), 1)
    brow = jax.lax.broadcasted_iota(jnp.int32, (_BATCH, _R), 0)
    G = (lane == brow * _W + (_W - 1)).astype(f32)
    feat = jnp.dot(G, x, precision=jax.lax.Precision.HIGHEST,
                   preferred_element_type=f32)
    feat_ref[...] = feat
    critic_ref[...] = _mdot(feat, Wc[...]) + bc[...]


def kernel(nodes, W_msg0, b_msg0, att0, W_self0, b_self0,
           W_msg1, b_msg1, att1,
           W_msg2, b_msg2, att2, W_self2, b_self2,
           W_msg3, b_msg3, att3,
           W_msg4, b_msg4, att4, W_self4, b_self4,
           W_critic, b_critic):
    f32 = jnp.float32
    x0 = nodes[:, _NUM_NODES - _W:].reshape(_R, 1).astype(f32)
    args = [
        x0,
        W_msg0, b_msg0.reshape(1, -1), att0.reshape(1, -1),
        W_self0, b_self0.reshape(1, -1),
        W_msg1, b_msg1.reshape(1, -1), att1.reshape(1, -1),
        W_msg2, b_msg2.reshape(1, -1), att2.reshape(1, -1),
        W_self2, b_self2.reshape(1, -1),
        W_msg3, b_msg3.reshape(1, -1), att3.reshape(1, -1),
        W_msg4, b_msg4.reshape(1, -1), att4.reshape(1, -1),
        W_self4, b_self4.reshape(1, -1),
        W_critic, b_critic.reshape(1, 1),
    ]
    critic, feat = pl.pallas_call(
        _body,
        out_shape=(
            jax.ShapeDtypeStruct((_BATCH, 1), f32),
            jax.ShapeDtypeStruct((_BATCH, _HIDDEN), f32),
        ),
    )(*args)
    return critic, feat
